# Initial kernel scaffold; baseline (speedup 1.0000x reference)
import jax
import jax.numpy as jnp
from jax.experimental import pallas as pl


def kernel(x, edge_index, edge_attr, Wl1, bl1, Wr1, g1, be1, Wl2, bl2, Wr2, g2, be2, W1, B1, W2, B2, W3, B3):
    raise NotImplementedError("write your pallas kernel here")


# speedup vs baseline: 3.8373x; 3.8373x over previous
import functools

import jax
import jax.numpy as jnp
from jax import lax
from jax.experimental import pallas as pl
from jax.experimental.pallas import tpu as pltpu
from jax.experimental.pallas import tpu_sc as plsc

_NC, _NS = 2, 16
_CH = 128
_NB = 1000
_EB = 4000

_HIGH = jax.lax.Precision.HIGHEST


def _dot(a, b, precision=_HIGH):
    return jnp.dot(a, b, preferred_element_type=jnp.float32,
                   precision=precision)


def _pre1_body(x_ref, wl_ref, wr_ref, bl_ref, t_ref, xr_ref):
    x = x_ref[...]
    t_ref[...] = _dot(x, wl_ref[...])
    xr_ref[...] = _dot(x, wr_ref[...], jax.lax.Precision.DEFAULT) + bl_ref[...]


def _stats_body(pa_ref, pb_ref, d0_ref, d1_ref, xr_ref, pre_ref, s_ref, q_ref):
    i = pl.program_id(0)
    deg = d0_ref[:, 0:1] + d1_ref[:, 0:1]
    inv = 1.0 / jnp.maximum(deg, 1.0)
    pre = (pa_ref[...] + pb_ref[...]) * inv + xr_ref[...]
    pre_ref[...] = pre
    bs = jnp.sum(pre, axis=0, keepdims=True)
    bq = jnp.sum(pre * pre, axis=0, keepdims=True)

    @pl.when(i == 0)
    def _():
        s_ref[...] = bs
        q_ref[...] = bq

    @pl.when(i != 0)
    def _():
        s_ref[...] += bs
        q_ref[...] += bq


def _bn_relu(pre_ref, s_ref, q_ref, g_ref, be_ref, n):
    mu = s_ref[...] * (1.0 / n)
    var = q_ref[...] * (1.0 / n) - mu * mu
    h = (pre_ref[...] - mu) * lax.rsqrt(var + 1e-5) * g_ref[...] + be_ref[...]
    return jnp.maximum(h, 0.0)


def _apply1_body(pre_ref, s_ref, q_ref, g_ref, be_ref, wl_ref, wr_ref, bl_ref,
                 t_ref, xr_ref, *, n):
    h = _bn_relu(pre_ref, s_ref, q_ref, g_ref, be_ref, n)
    t_ref[...] = _dot(h, wl_ref[...])
    xr_ref[...] = _dot(h, wr_ref[...], jax.lax.Precision.DEFAULT) + bl_ref[...]


def _apply2_body(pre_ref, s_ref, q_ref, g_ref, be_ref, h_ref, *, n):
    h_ref[...] = _bn_relu(pre_ref, s_ref, q_ref, g_ref, be_ref, n)


def _mlp_body(hs_ref, hr_ref, ea_ref, w1s_ref, w1r_ref, w1e_ref, b1_ref,
              w2_ref, b2_ref, w3_ref, b3_ref, o_ref):
    p = jax.lax.Precision.DEFAULT
    z = _dot(hs_ref[...], w1s_ref[...], p)
    z += _dot(hr_ref[...], w1r_ref[...], p)
    z += _dot(ea_ref[...], w1e_ref[...], p)
    z = jnp.maximum(z + b1_ref[...], 0.0)
    z = jnp.maximum(_dot(z, w2_ref[...], p) + b2_ref[...], 0.0)
    o_ref[...] = _dot(z, w3_ref[...], p) + b3_ref[...]


def _full(shape):
    return pl.BlockSpec(shape, lambda i: (0,) * len(shape))


def _rows(shape):
    return pl.BlockSpec(shape, lambda i: (i,) + (0,) * (len(shape) - 1))


def _sc_mesh():
    return plsc.VectorSubcoreMesh(core_axis_name="c", subcore_axis_name="s",
                                  num_cores=_NC, num_subcores=_NS)


_SC_PARAMS = pltpu.CompilerParams(use_tc_tiling_on_sc=False)


def _sc_segsum(table, srcr, dstr, z64, z16):
    n, d = table.shape
    nch, _, ch = srcr.shape
    with_deg = z16 is not None
    per_core = nch // _NC
    kmax = (per_core + _NS - 1) // _NS
    rpw = (n // (_NS * 8)) * 8
    rem = n - rpw * _NS

    out_type = [jax.ShapeDtypeStruct((_NC, n, d), jnp.float32)]
    scratch = [
        pltpu.VMEM((1, ch), jnp.int32),
        pltpu.VMEM((1, ch), jnp.int32),
        pltpu.VMEM((ch, d), jnp.float32),
        pltpu.VMEM_SHARED((n, d), jnp.float32),
    ]
    if with_deg:
        out_type.append(jax.ShapeDtypeStruct((_NC, n, 16), jnp.float32))
        scratch += [
            pltpu.VMEM((ch, 16), jnp.float32),
            pltpu.VMEM_SHARED((n, 16), jnp.float32),
        ]

    def body(*refs):
        if with_deg:
            (tbl, sr, dr, z64r, z16r, out, dout,
             sidx, didx, rows, acc, ones_v, dacc) = refs
        else:
            (tbl, sr, dr, z64r, out, sidx, didx, rows, acc) = refs
        cid = lax.axis_index("c")
        sid = lax.axis_index("s")
        row0 = pl.multiple_of(sid * rpw, 8)
        pltpu.sync_copy(z64r.at[pl.ds(row0, rpw)], acc.at[pl.ds(row0, rpw)])
        if with_deg:
            pltpu.sync_copy(z16r.at[pl.ds(row0, rpw)], dacc.at[pl.ds(row0, rpw)])

            @pl.loop(0, ch)
            def _(i):
                ones_v[i, :] = jnp.ones((16,), jnp.float32)

        if rem:
            @pl.when(sid == 0)
            def _():
                pltpu.sync_copy(z64r.at[pl.ds(n - rem, rem)],
                                acc.at[pl.ds(n - rem, rem)])
                if with_deg:
                    pltpu.sync_copy(z16r.at[pl.ds(n - rem, rem)],
                                    dacc.at[pl.ds(n - rem, rem)])

        plsc.subcore_barrier()

        @pl.loop(0, kmax)
        def _(k):
            j = k * _NS + sid

            @pl.when(j < per_core)
            def _():
                g = cid * per_core + j
                pltpu.sync_copy(sr.at[g], sidx)
                pltpu.sync_copy(dr.at[g], didx)
                pltpu.sync_copy(tbl.at[sidx.at[0]], rows)
                pltpu.sync_copy(rows, acc.at[didx.at[0]], add=True)
                if with_deg:
                    pltpu.sync_copy(ones_v, dacc.at[didx.at[0]], add=True)

        plsc.subcore_barrier()
        pltpu.sync_copy(acc.at[pl.ds(row0, rpw)], out.at[cid, pl.ds(row0, rpw)])
        if with_deg:
            pltpu.sync_copy(dacc.at[pl.ds(row0, rpw)],
                            dout.at[cid, pl.ds(row0, rpw)])
        if rem:
            @pl.when(sid == 0)
            def _():
                pltpu.sync_copy(acc.at[pl.ds(n - rem, rem)],
                                out.at[cid, pl.ds(n - rem, rem)])
                if with_deg:
                    pltpu.sync_copy(dacc.at[pl.ds(n - rem, rem)],
                                    dout.at[cid, pl.ds(n - rem, rem)])

    fn = pl.kernel(body, out_type=tuple(out_type), mesh=_sc_mesh(),
                   scratch_types=tuple(scratch), compiler_params=_SC_PARAMS)
    args = (table, srcr, dstr, z64) + ((z16,) if with_deg else ())
    return fn(*args)


def _sc_gather(table, srcr, dstr):
    n, d = table.shape
    nch, _, ch = srcr.shape
    nw = _NC * _NS
    kmax = (nch + nw - 1) // nw

    out_type = (jax.ShapeDtypeStruct((nch * ch, d), jnp.float32),
                jax.ShapeDtypeStruct((nch * ch, d), jnp.float32))
    scratch = (
        pltpu.VMEM((1, ch), jnp.int32),
        pltpu.VMEM((1, ch), jnp.int32),
        pltpu.VMEM((ch, d), jnp.float32),
        pltpu.VMEM((ch, d), jnp.float32),
    )

    def body(tbl, sr, dr, hs, hr, sidx, didx, rows_s, rows_d):
        w = lax.axis_index("c") * _NS + lax.axis_index("s")

        @pl.loop(0, kmax)
        def _(k):
            g = k * nw + w

            @pl.when(g < nch)
            def _():
                e0 = pl.multiple_of(g * ch, 8)
                pltpu.sync_copy(sr.at[g], sidx)
                pltpu.sync_copy(tbl.at[sidx.at[0]], rows_s)
                pltpu.sync_copy(rows_s, hs.at[pl.ds(e0, ch)])
                pltpu.sync_copy(dr.at[g], didx)
                pltpu.sync_copy(tbl.at[didx.at[0]], rows_d)
                pltpu.sync_copy(rows_d, hr.at[pl.ds(e0, ch)])

    fn = pl.kernel(body, out_type=out_type, mesh=_sc_mesh(),
                   scratch_types=scratch, compiler_params=_SC_PARAMS)
    return fn(table, srcr, dstr)


def kernel(x, edge_index, edge_attr, Wl1, bl1, Wr1, g1, be1, Wl2, bl2, Wr2,
           g2, be2, W1, B1, W2, B2, W3, B3):
    n, df = x.shape
    e = edge_index.shape[1]
    h = Wl1.shape[0]
    de = edge_attr.shape[1]
    f32 = jnp.float32

    srcr = edge_index[0].reshape(e // _CH, 1, _CH)
    dstr = edge_index[1].reshape(e // _CH, 1, _CH)
    z64 = jnp.zeros((n, h), f32)
    z16 = jnp.zeros((n, 16), f32)
    row = lambda v: v.reshape(1, -1)
    gn = n // _NB

    t1, xr1 = pl.pallas_call(
        _pre1_body,
        grid=(gn,),
        in_specs=[_rows((_NB, df)), _full((df, h)), _full((df, h)),
                  _full((1, h))],
        out_specs=[_rows((_NB, h)), _rows((_NB, h))],
        out_shape=[jax.ShapeDtypeStruct((n, h), f32)] * 2,
    )(x, Wl1.T, Wr1.T, row(bl1))

    p1, pdeg = _sc_segsum(t1, srcr, dstr, z64, z16)

    stats_call = pl.pallas_call(
        _stats_body,
        grid=(gn,),
        in_specs=[_rows((_NB, h)), _rows((_NB, h)), _rows((_NB, 16)),
                  _rows((_NB, 16)), _rows((_NB, h))],
        out_specs=[_rows((_NB, h)), _full((1, h)), _full((1, h))],
        out_shape=[jax.ShapeDtypeStruct((n, h), f32),
                   jax.ShapeDtypeStruct((1, h), f32),
                   jax.ShapeDtypeStruct((1, h), f32)],
    )

    pre1, s1, q1 = stats_call(p1[0], p1[1], pdeg[0], pdeg[1], xr1)

    t2, xr2 = pl.pallas_call(
        functools.partial(_apply1_body, n=float(n)),
        grid=(gn,),
        in_specs=[_rows((_NB, h)), _full((1, h)), _full((1, h)),
                  _full((1, h)), _full((1, h)), _full((h, h)), _full((h, h)),
                  _full((1, h))],
        out_specs=[_rows((_NB, h)), _rows((_NB, h))],
        out_shape=[jax.ShapeDtypeStruct((n, h), f32)] * 2,
    )(pre1, s1, q1, row(g1), row(be1), Wl2.T, Wr2.T, row(bl2))

    (p2,) = _sc_segsum(t2, srcr, dstr, z64, None)

    pre2, s2, q2 = stats_call(p2[0], p2[1], pdeg[0], pdeg[1], xr2)

    h2 = pl.pallas_call(
        functools.partial(_apply2_body, n=float(n)),
        grid=(gn,),
        in_specs=[_rows((_NB, h)), _full((1, h)), _full((1, h)),
                  _full((1, h)), _full((1, h))],
        out_specs=_rows((_NB, h)),
        out_shape=jax.ShapeDtypeStruct((n, h), f32),
    )(pre2, s2, q2, row(g2), row(be2))

    hs, hr = _sc_gather(h2, srcr, dstr)

    out = pl.pallas_call(
        _mlp_body,
        grid=(e // _EB,),
        in_specs=[_rows((_EB, h)), _rows((_EB, h)), _rows((_EB, de)),
                  _full((h, 128)), _full((h, 128)), _full((de, 128)),
                  _full((1, 128)), _full((128, 64)), _full((1, 64)),
                  _full((64, 2)), _full((1, 2))],
        out_specs=_rows((_EB, 2)),
        out_shape=jax.ShapeDtypeStruct((e, 2), f32),
    )(hs, hr, edge_attr, W1[:, :h].T, W1[:, h + de:].T, W1[:, h:h + de].T,
      row(B1), W2.T, row(B2), W3.T, row(B3))

    return out


# speedup vs baseline: 5.0022x; 1.3036x over previous
import functools

import jax
import jax.numpy as jnp
from jax import lax
from jax.experimental import pallas as pl
from jax.experimental.pallas import tpu as pltpu
from jax.experimental.pallas import tpu_sc as plsc

_NC, _NS = 2, 16
_CH = 128
_NB = 1000
_EB = 4000

_HIGH = jax.lax.Precision.HIGHEST


def _dot(a, b, precision=_HIGH):
    return jnp.dot(a, b, preferred_element_type=jnp.float32,
                   precision=precision)


def _pre1_body(x_ref, wl_ref, wr_ref, bl_ref, t_ref, xr_ref):
    x = x_ref[...]
    t_ref[...] = _dot(x, wl_ref[...])
    xr_ref[...] = _dot(x, wr_ref[...], jax.lax.Precision.DEFAULT) + bl_ref[...]


def _stats_body(p_ref, d_ref, xr_ref, pre_ref, s_ref, q_ref):
    i = pl.program_id(0)
    h = xr_ref.shape[1]
    deg = d_ref[:, 0:1] + d_ref[:, h:h + 1]
    inv = 1.0 / jnp.maximum(deg, 1.0)
    pre = (p_ref[:, :h] + p_ref[:, h:]) * inv + xr_ref[...]
    pre_ref[...] = pre
    bs = jnp.sum(pre, axis=0, keepdims=True)
    bq = jnp.sum(pre * pre, axis=0, keepdims=True)

    @pl.when(i == 0)
    def _():
        s_ref[...] = bs
        q_ref[...] = bq

    @pl.when(i != 0)
    def _():
        s_ref[...] += bs
        q_ref[...] += bq


def _bn_relu(pre_ref, s_ref, q_ref, g_ref, be_ref, n):
    mu = s_ref[...] * (1.0 / n)
    var = q_ref[...] * (1.0 / n) - mu * mu
    h = (pre_ref[...] - mu) * lax.rsqrt(var + 1e-5) * g_ref[...] + be_ref[...]
    return jnp.maximum(h, 0.0)


def _apply1_body(pre_ref, s_ref, q_ref, g_ref, be_ref, wl_ref, wr_ref, bl_ref,
                 t_ref, xr_ref, *, n):
    h = _bn_relu(pre_ref, s_ref, q_ref, g_ref, be_ref, n)
    t_ref[...] = _dot(h, wl_ref[...])
    xr_ref[...] = _dot(h, wr_ref[...], jax.lax.Precision.DEFAULT) + bl_ref[...]


def _apply2_body(pre_ref, s_ref, q_ref, g_ref, be_ref, h_ref, *, n):
    h_ref[...] = _bn_relu(pre_ref, s_ref, q_ref, g_ref, be_ref, n)


def _mlp_body(hsr_ref, ea_ref, w1sr_ref, w1e_ref, b1_ref,
              w2_ref, b2_ref, w3_ref, b3_ref, o_ref):
    p = jax.lax.Precision.DEFAULT
    z = _dot(hsr_ref[...], w1sr_ref[...], p)
    z += _dot(ea_ref[...], w1e_ref[...], p)
    z = jnp.maximum(z + b1_ref[...], 0.0)
    z = jnp.maximum(_dot(z, w2_ref[...], p) + b2_ref[...], 0.0)
    o_ref[...] = _dot(z, w3_ref[...], p) + b3_ref[...]


def _full(shape):
    return pl.BlockSpec(shape, lambda i: (0,) * len(shape))


def _rows(shape):
    return pl.BlockSpec(shape, lambda i: (i,) + (0,) * (len(shape) - 1))


def _sc_mesh():
    return plsc.VectorSubcoreMesh(core_axis_name="c", subcore_axis_name="s",
                                  num_cores=_NC, num_subcores=_NS)


_SC_PARAMS = pltpu.CompilerParams(use_tc_tiling_on_sc=False)


def _sc_segsum(table, srcr, dstr, z64, z16):
    n, d = table.shape
    nch, _, ch = srcr.shape
    with_deg = z16 is not None
    per_core = nch // _NC
    kmax = (per_core + _NS - 1) // _NS
    rpw = (n // (_NS * 8)) * 8
    rem = n - rpw * _NS

    out_type = [jax.ShapeDtypeStruct((n, 2 * d), jnp.float32)]
    scratch = [
        pltpu.VMEM((1, ch), jnp.int32),
        pltpu.VMEM((1, ch), jnp.int32),
        pltpu.VMEM((ch, d), jnp.float32),
        pltpu.VMEM_SHARED((n, d), jnp.float32),
    ]
    if with_deg:
        out_type.append(jax.ShapeDtypeStruct((n, 2 * d), jnp.float32))
        scratch += [
            pltpu.VMEM((ch, 16), jnp.float32),
            pltpu.VMEM_SHARED((n, 16), jnp.float32),
        ]

    def body(*refs):
        if with_deg:
            (tbl, sr, dr, z64r, z16r, out, dout,
             sidx, didx, rows, acc, ones_v, dacc) = refs
        else:
            (tbl, sr, dr, z64r, out, sidx, didx, rows, acc) = refs
        cid = lax.axis_index("c")
        sid = lax.axis_index("s")
        row0 = pl.multiple_of(sid * rpw, 8)
        pltpu.sync_copy(z64r.at[pl.ds(row0, rpw)], acc.at[pl.ds(row0, rpw)])
        if with_deg:
            pltpu.sync_copy(z16r.at[pl.ds(row0, rpw)], dacc.at[pl.ds(row0, rpw)])

            @pl.loop(0, ch)
            def _(i):
                ones_v[i, :] = jnp.ones((16,), jnp.float32)

        if rem:
            @pl.when(sid == 0)
            def _():
                pltpu.sync_copy(z64r.at[pl.ds(n - rem, rem)],
                                acc.at[pl.ds(n - rem, rem)])
                if with_deg:
                    pltpu.sync_copy(z16r.at[pl.ds(n - rem, rem)],
                                    dacc.at[pl.ds(n - rem, rem)])

        plsc.subcore_barrier()

        @pl.loop(0, kmax)
        def _(k):
            j = k * _NS + sid

            @pl.when(j < per_core)
            def _():
                g = cid * per_core + j
                pltpu.sync_copy(sr.at[g], sidx)
                pltpu.sync_copy(dr.at[g], didx)
                pltpu.sync_copy(tbl.at[sidx.at[0]], rows)
                pltpu.sync_copy(rows, acc.at[didx.at[0]], add=True)
                if with_deg:
                    pltpu.sync_copy(ones_v, dacc.at[didx.at[0]], add=True)

        plsc.subcore_barrier()
        col0 = cid * d
        pltpu.sync_copy(acc.at[pl.ds(row0, rpw)],
                        out.at[pl.ds(row0, rpw), pl.ds(col0, d)])
        if with_deg:
            pltpu.sync_copy(dacc.at[pl.ds(row0, rpw)],
                            dout.at[pl.ds(row0, rpw), pl.ds(col0, 16)])
        if rem:
            @pl.when(sid == 0)
            def _():
                pltpu.sync_copy(acc.at[pl.ds(n - rem, rem)],
                                out.at[pl.ds(n - rem, rem), pl.ds(col0, d)])
                if with_deg:
                    pltpu.sync_copy(dacc.at[pl.ds(n - rem, rem)],
                                    dout.at[pl.ds(n - rem, rem), pl.ds(col0, 16)])

    fn = pl.kernel(body, out_type=tuple(out_type), mesh=_sc_mesh(),
                   scratch_types=tuple(scratch), compiler_params=_SC_PARAMS)
    args = (table, srcr, dstr, z64) + ((z16,) if with_deg else ())
    return fn(*args)


def _sc_gather(table, srcr, dstr):
    n, d = table.shape
    nch, _, ch = srcr.shape
    nw = _NC * _NS
    kmax = (nch + nw - 1) // nw

    out_type = jax.ShapeDtypeStruct((nch * ch, 2 * d), jnp.float32)
    scratch = (
        pltpu.VMEM((1, ch), jnp.int32),
        pltpu.VMEM((1, ch), jnp.int32),
        pltpu.VMEM((ch, d), jnp.float32),
        pltpu.VMEM((ch, d), jnp.float32),
    )

    def body(tbl, sr, dr, hsr, sidx, didx, rows_s, rows_d):
        w = lax.axis_index("c") * _NS + lax.axis_index("s")

        @pl.loop(0, kmax)
        def _(k):
            g = k * nw + w

            @pl.when(g < nch)
            def _():
                e0 = pl.multiple_of(g * ch, 8)
                pltpu.sync_copy(sr.at[g], sidx)
                pltpu.sync_copy(tbl.at[sidx.at[0]], rows_s)
                pltpu.sync_copy(rows_s, hsr.at[pl.ds(e0, ch), pl.ds(0, d)])
                pltpu.sync_copy(dr.at[g], didx)
                pltpu.sync_copy(tbl.at[didx.at[0]], rows_d)
                pltpu.sync_copy(rows_d, hsr.at[pl.ds(e0, ch), pl.ds(d, d)])

    fn = pl.kernel(body, out_type=out_type, mesh=_sc_mesh(),
                   scratch_types=scratch, compiler_params=_SC_PARAMS)
    return fn(table, srcr, dstr)


def kernel(x, edge_index, edge_attr, Wl1, bl1, Wr1, g1, be1, Wl2, bl2, Wr2,
           g2, be2, W1, B1, W2, B2, W3, B3):
    n, df = x.shape
    e = edge_index.shape[1]
    h = Wl1.shape[0]
    de = edge_attr.shape[1]
    f32 = jnp.float32

    srcr = edge_index[0].reshape(e // _CH, 1, _CH)
    dstr = edge_index[1].reshape(e // _CH, 1, _CH)
    z64 = jnp.zeros((n, h), f32)
    z16 = jnp.zeros((n, 16), f32)
    row = lambda v: v.reshape(1, -1)
    gn = n // _NB

    t1, xr1 = pl.pallas_call(
        _pre1_body,
        grid=(gn,),
        in_specs=[_rows((_NB, df)), _full((df, h)), _full((df, h)),
                  _full((1, h))],
        out_specs=[_rows((_NB, h)), _rows((_NB, h))],
        out_shape=[jax.ShapeDtypeStruct((n, h), f32)] * 2,
    )(x, Wl1.T, Wr1.T, row(bl1))

    p1, pdeg = _sc_segsum(t1, srcr, dstr, z64, z16)

    stats_call = pl.pallas_call(
        _stats_body,
        grid=(gn,),
        in_specs=[_rows((_NB, 2 * h)), _rows((_NB, 2 * h)), _rows((_NB, h))],
        out_specs=[_rows((_NB, h)), _full((1, h)), _full((1, h))],
        out_shape=[jax.ShapeDtypeStruct((n, h), f32),
                   jax.ShapeDtypeStruct((1, h), f32),
                   jax.ShapeDtypeStruct((1, h), f32)],
    )

    pre1, s1, q1 = stats_call(p1, pdeg, xr1)

    t2, xr2 = pl.pallas_call(
        functools.partial(_apply1_body, n=float(n)),
        grid=(gn,),
        in_specs=[_rows((_NB, h)), _full((1, h)), _full((1, h)),
                  _full((1, h)), _full((1, h)), _full((h, h)), _full((h, h)),
                  _full((1, h))],
        out_specs=[_rows((_NB, h)), _rows((_NB, h))],
        out_shape=[jax.ShapeDtypeStruct((n, h), f32)] * 2,
    )(pre1, s1, q1, row(g1), row(be1), Wl2.T, Wr2.T, row(bl2))

    (p2,) = _sc_segsum(t2, srcr, dstr, z64, None)

    pre2, s2, q2 = stats_call(p2, pdeg, xr2)

    h2 = pl.pallas_call(
        functools.partial(_apply2_body, n=float(n)),
        grid=(gn,),
        in_specs=[_rows((_NB, h)), _full((1, h)), _full((1, h)),
                  _full((1, h)), _full((1, h))],
        out_specs=_rows((_NB, h)),
        out_shape=jax.ShapeDtypeStruct((n, h), f32),
    )(pre2, s2, q2, row(g2), row(be2))

    hsr = _sc_gather(h2, srcr, dstr)

    w1sr = jnp.concatenate([W1[:, :h].T, W1[:, h + de:].T], axis=0)
    out = pl.pallas_call(
        _mlp_body,
        grid=(e // _EB,),
        in_specs=[_rows((_EB, 2 * h)), _rows((_EB, de)),
                  _full((2 * h, 128)), _full((de, 128)),
                  _full((1, 128)), _full((128, 64)), _full((1, 64)),
                  _full((64, 2)), _full((1, 2))],
        out_specs=_rows((_EB, 2)),
        out_shape=jax.ShapeDtypeStruct((e, 2), f32),
    )(hsr, edge_attr, w1sr, W1[:, h:h + de].T,
      row(B1), W2.T, row(B2), W3.T, row(B3))

    return out
